# Initial kernel scaffold; baseline (speedup 1.0000x reference)
import jax
import jax.numpy as jnp
from jax.experimental import pallas as pl


def kernel(x3, edge_index, W1, Wl1, b1, Wb, Wlb, bb, W2, Wl2, b2, Wg, Wlg, bg):
    raise NotImplementedError("write your pallas kernel here")


# speedup vs baseline: 3.0032x; 3.0032x over previous
import functools

import jax
import jax.numpy as jnp
from jax import lax
from jax.experimental import pallas as pl
from jax.experimental.pallas import tpu as pltpu
from jax.experimental.pallas import tpu_sc as plsc

N = 10000
E = 320000
NC = 2
NS = 16
NW = NC * NS
EW = E // NW
CH = 80
NCHUNK = EW // CH
NP = 10240
ROWS_PER_TILE = NP // NS


_MESH = plsc.VectorSubcoreMesh(core_axis_name="c", subcore_axis_name="s",
                               num_cores=NC, num_subcores=NS)
_SC_PARAMS = pltpu.CompilerParams(use_tc_tiling_on_sc=False)

HC = 96
ET = E // NS
NCH_T = ET // CH


@functools.partial(
    pl.kernel,
    out_type=jax.ShapeDtypeStruct((NC, NP, HC), jnp.float32),
    mesh=_MESH,
    scratch_types=[
        pltpu.VMEM((CH,), jnp.int32),
        pltpu.VMEM((CH,), jnp.int32),
        pltpu.VMEM((CH, HC), jnp.float32),
        pltpu.VMEM_SHARED((NP, HC), jnp.float32),
        pltpu.SemaphoreType.DMA,
    ],
    compiler_params=_SC_PARAMS,
)
def _segsum192(table, src, dst, zeros, out, src_v, dst_v, rows_v, agg, sem):
    cid = lax.axis_index("c")
    sid = lax.axis_index("s")

    r0 = sid * ROWS_PER_TILE
    pltpu.sync_copy(zeros.at[pl.ds(r0, ROWS_PER_TILE)],
                    agg.at[pl.ds(r0, ROWS_PER_TILE)])
    plsc.subcore_barrier()

    def body(i, carry):
        base = sid * ET + i * CH
        pltpu.sync_copy(src.at[pl.ds(base, CH)], src_v)
        pltpu.sync_copy(dst.at[pl.ds(base, CH)], dst_v)
        pltpu.async_copy(table.at[cid].at[src_v], rows_v, sem).wait()
        pltpu.sync_copy(rows_v, agg.at[dst_v], add=True)
        return carry

    lax.fori_loop(0, NCH_T, body, 0)
    plsc.subcore_barrier()
    pltpu.sync_copy(agg.at[pl.ds(r0, ROWS_PER_TILE)],
                    out.at[cid, pl.ds(r0, ROWS_PER_TILE)])


@functools.partial(
    pl.kernel,
    out_type=jax.ShapeDtypeStruct((NC, NP, 16), jnp.float32),
    mesh=_MESH,
    scratch_types=[
        pltpu.VMEM((CH,), jnp.int32),
        pltpu.VMEM((CH,), jnp.int32),
        pltpu.VMEM((CH, 16), jnp.float32),
        pltpu.VMEM_SHARED((NP, 16), jnp.float32),
        pltpu.SemaphoreType.DMA,
    ],
    compiler_params=_SC_PARAMS,
)
def _segsum16(table, src, dst, zeros, out, src_v, dst_v, rows_v, agg, sem):
    cid = lax.axis_index("c")
    sid = lax.axis_index("s")
    wid = sid * NC + cid

    r0 = sid * ROWS_PER_TILE
    pltpu.sync_copy(zeros.at[pl.ds(r0, ROWS_PER_TILE)],
                    agg.at[pl.ds(r0, ROWS_PER_TILE)])
    plsc.subcore_barrier()

    def body(i, carry):
        base = wid * EW + i * CH
        pltpu.sync_copy(src.at[pl.ds(base, CH)], src_v)
        pltpu.sync_copy(dst.at[pl.ds(base, CH)], dst_v)
        pltpu.async_copy(table.at[src_v], rows_v, sem).wait()
        pltpu.sync_copy(rows_v, agg.at[dst_v], add=True)
        return carry

    lax.fori_loop(0, NCHUNK, body, 0)
    plsc.subcore_barrier()
    pltpu.sync_copy(agg.at[pl.ds(r0, ROWS_PER_TILE)],
                    out.at[cid, pl.ds(r0, ROWS_PER_TILE)])


def _xw2_split_body(x_ref, w_ref, wl_ref, b_ref, s_ref, l_ref):
    x = x_ref[...]
    s = jnp.dot(x, w_ref[...], preferred_element_type=jnp.float32)
    s_ref[0] = s[:, :HC]
    s_ref[1] = s[:, HC:]
    l_ref[...] = jnp.dot(x, wl_ref[...], preferred_element_type=jnp.float32) + b_ref[...]


def _xw2_split(x, w, wl, b, bn=2000):
    n, cin = x.shape
    co = w.shape[1]
    grid = (n // bn,)
    return pl.pallas_call(
        _xw2_split_body,
        grid=grid,
        in_specs=[
            pl.BlockSpec((bn, cin), lambda i: (i, 0)),
            pl.BlockSpec((cin, co), lambda i: (0, 0)),
            pl.BlockSpec((cin, co), lambda i: (0, 0)),
            pl.BlockSpec((1, co), lambda i: (0, 0)),
        ],
        out_specs=[
            pl.BlockSpec((NC, bn, HC), lambda i: (0, i, 0)),
            pl.BlockSpec((bn, co), lambda i: (i, 0)),
        ],
        out_shape=[
            jax.ShapeDtypeStruct((NC, n, HC), jnp.float32),
            jax.ShapeDtypeStruct((n, co), jnp.float32),
        ],
    )(x, w, wl, b.reshape(1, co))


def _xw2_plain_body(x_ref, w_ref, wl_ref, b_ref, s_ref, l_ref):
    x = x_ref[...]
    s_ref[...] = jnp.dot(x, w_ref[...], preferred_element_type=jnp.float32)
    l_ref[...] = jnp.dot(x, wl_ref[...], preferred_element_type=jnp.float32) + b_ref[...]


def _xw2_plain(x, w, wl, b, bn=2000):
    n, cin = x.shape
    co = w.shape[1]
    grid = (n // bn,)
    return pl.pallas_call(
        _xw2_plain_body,
        grid=grid,
        in_specs=[
            pl.BlockSpec((bn, cin), lambda i: (i, 0)),
            pl.BlockSpec((cin, co), lambda i: (0, 0)),
            pl.BlockSpec((cin, co), lambda i: (0, 0)),
            pl.BlockSpec((1, co), lambda i: (0, 0)),
        ],
        out_specs=[
            pl.BlockSpec((bn, co), lambda i: (i, 0)),
            pl.BlockSpec((bn, co), lambda i: (i, 0)),
        ],
        out_shape=[
            jax.ShapeDtypeStruct((n, co), jnp.float32),
            jax.ShapeDtypeStruct((n, co), jnp.float32),
        ],
    )(x, w, wl, b.reshape(1, co))


def _merge_relu_body(z0_ref, z1_ref, l_ref, o_ref):
    z = jnp.concatenate([z0_ref[...], z1_ref[...]], axis=1)
    o_ref[...] = jnp.maximum(z + l_ref[...], 0.0)


def _merge_res_body(z0_ref, z1_ref, l_ref, h_ref, o_ref):
    z = jnp.concatenate([z0_ref[...], z1_ref[...]], axis=1)
    y = jnp.maximum(z + l_ref[...], 0.0)
    o_ref[...] = (h_ref[...] + y) * 0.5


def _merge_add_body(z0_ref, z1_ref, l_ref, o_ref):
    o_ref[...] = z0_ref[...] + z1_ref[...] + l_ref[...]


def _merge(body, arrs, co, bn=2000):
    n = arrs[0].shape[0]
    grid = (n // bn,)
    return pl.pallas_call(
        body,
        grid=grid,
        in_specs=[pl.BlockSpec((bn, a.shape[1]), lambda i: (i, 0)) for a in arrs],
        out_specs=pl.BlockSpec((bn, co), lambda i: (i, 0)),
        out_shape=jax.ShapeDtypeStruct((n, co), jnp.float32),
    )(*arrs)


def kernel(x3, edge_index, W1, Wl1, b1, Wb, Wlb, bb, W2, Wl2, b2, Wg, Wlg, bg):
    x = x3[0]
    src = edge_index[0]
    dst = edge_index[1]
    zeros96 = jnp.zeros((NP, HC), jnp.float32)
    zeros16 = jnp.zeros((NP, 16), jnp.float32)

    def gconv_relu(xin, w, wl, b):
        s2, l = _xw2_split(xin, w, wl, b)
        parts = _segsum192(s2, src, dst, zeros96)
        return _merge(_merge_relu_body, (parts[0, :N], parts[1, :N], l), 192)

    def gconv_res(xin, w, wl, b, h):
        s2, l = _xw2_split(xin, w, wl, b)
        parts = _segsum192(s2, src, dst, zeros96)
        return _merge(_merge_res_body, (parts[0, :N], parts[1, :N], l, h), 192)

    h = gconv_relu(x, W1, Wl1, b1)
    for i in range(6):
        y = gconv_relu(h, Wb[2 * i], Wlb[2 * i], bb[2 * i])
        h = gconv_res(y, Wb[2 * i + 1], Wlb[2 * i + 1], bb[2 * i + 1], h)
    xo = gconv_relu(h, W2, Wl2, b2)
    wg_p = jnp.pad(Wg, ((0, 0), (0, 13)))
    wlg_p = jnp.pad(Wlg, ((0, 0), (0, 13)))
    bg_p = jnp.pad(bg, (0, 13))
    sg, lg = _xw2_plain(xo, wg_p, wlg_p, bg_p)
    parts = _segsum16(sg, src, dst, zeros16)
    out16 = _merge(_merge_add_body, (parts[0, :N], parts[1, :N], lg), 16)
    return out16[:, :3][None]


# speedup vs baseline: 8.2991x; 2.7635x over previous
import functools

import jax
import jax.numpy as jnp
from jax import lax
from jax.experimental import pallas as pl
from jax.experimental.pallas import tpu as pltpu
from jax.experimental.pallas import tpu_sc as plsc

N = 10000
E = 320000
NC = 2
NS = 16
NW = NC * NS
EW = E // NW
CH = 80
NCHUNK = EW // CH
NP = 10240
ROWS_PER_TILE = NP // NS


_MESH = plsc.VectorSubcoreMesh(core_axis_name="c", subcore_axis_name="s",
                               num_cores=NC, num_subcores=NS)
_SC_PARAMS = pltpu.CompilerParams(use_tc_tiling_on_sc=False)

HC = 96
ET = E // NS
CH2 = 125
NCH2 = ET // CH2
NPAIR = NCH2 // 2


@functools.partial(
    pl.kernel,
    out_type=jax.ShapeDtypeStruct((NC, NP, HC), jnp.float32),
    mesh=_MESH,
    scratch_types=[
        pltpu.VMEM((NCH2, CH2), jnp.int32),
        pltpu.VMEM((NCH2, CH2), jnp.int32),
        pltpu.VMEM((CH2, HC), jnp.float32),
        pltpu.VMEM((CH2, HC), jnp.float32),
        pltpu.VMEM_SHARED((NP, HC), jnp.float32),
        pltpu.SemaphoreType.DMA,
        pltpu.SemaphoreType.DMA,
    ],
    compiler_params=_SC_PARAMS,
)
def _segsum192(table, src3, dst3, zeros, out,
               src_s, dst_s, rows0, rows1, agg, sem0, sem1):
    cid = lax.axis_index("c")
    sid = lax.axis_index("s")

    r0 = sid * ROWS_PER_TILE
    pltpu.sync_copy(zeros.at[pl.ds(r0, ROWS_PER_TILE)],
                    agg.at[pl.ds(r0, ROWS_PER_TILE)])
    pltpu.sync_copy(src3.at[sid], src_s)
    pltpu.sync_copy(dst3.at[sid], dst_s)
    plsc.subcore_barrier()

    tab = table.at[cid]
    pltpu.async_copy(tab.at[src_s.at[0]], rows0, sem0)

    def body(j, carry):
        i0 = 2 * j
        pltpu.async_copy(tab.at[src_s.at[i0 + 1]], rows1, sem1)
        pltpu.make_async_copy(tab.at[src_s.at[0]], rows0, sem0).wait()
        pltpu.sync_copy(rows0, agg.at[dst_s.at[i0]], add=True)
        nxt = lax.rem(i0 + 2, NCH2)
        pltpu.async_copy(tab.at[src_s.at[nxt]], rows0, sem0)
        pltpu.make_async_copy(tab.at[src_s.at[0]], rows1, sem1).wait()
        pltpu.sync_copy(rows1, agg.at[dst_s.at[i0 + 1]], add=True)
        return carry

    lax.fori_loop(0, NPAIR, body, 0)
    pltpu.make_async_copy(tab.at[src_s.at[0]], rows0, sem0).wait()
    plsc.subcore_barrier()
    pltpu.sync_copy(agg.at[pl.ds(r0, ROWS_PER_TILE)],
                    out.at[cid, pl.ds(r0, ROWS_PER_TILE)])


NCH16 = EW // CH2
NPAIR16 = NCH16 // 2


@functools.partial(
    pl.kernel,
    out_type=jax.ShapeDtypeStruct((NC, NP, 16), jnp.float32),
    mesh=_MESH,
    scratch_types=[
        pltpu.VMEM((NCH16, CH2), jnp.int32),
        pltpu.VMEM((NCH16, CH2), jnp.int32),
        pltpu.VMEM((CH2, 16), jnp.float32),
        pltpu.VMEM((CH2, 16), jnp.float32),
        pltpu.VMEM_SHARED((NP, 16), jnp.float32),
        pltpu.SemaphoreType.DMA,
        pltpu.SemaphoreType.DMA,
    ],
    compiler_params=_SC_PARAMS,
)
def _segsum16(table, src4, dst4, zeros, out,
              src_s, dst_s, rows0, rows1, agg, sem0, sem1):
    cid = lax.axis_index("c")
    sid = lax.axis_index("s")
    wid = sid * NC + cid

    r0 = sid * ROWS_PER_TILE
    pltpu.sync_copy(zeros.at[pl.ds(r0, ROWS_PER_TILE)],
                    agg.at[pl.ds(r0, ROWS_PER_TILE)])
    pltpu.sync_copy(src4.at[wid], src_s)
    pltpu.sync_copy(dst4.at[wid], dst_s)
    plsc.subcore_barrier()

    pltpu.async_copy(table.at[src_s.at[0]], rows0, sem0)

    def body(j, carry):
        i0 = 2 * j
        pltpu.async_copy(table.at[src_s.at[i0 + 1]], rows1, sem1)
        pltpu.make_async_copy(table.at[src_s.at[0]], rows0, sem0).wait()
        pltpu.sync_copy(rows0, agg.at[dst_s.at[i0]], add=True)
        nxt = lax.rem(i0 + 2, NCH16)
        pltpu.async_copy(table.at[src_s.at[nxt]], rows0, sem0)
        pltpu.make_async_copy(table.at[src_s.at[0]], rows1, sem1).wait()
        pltpu.sync_copy(rows1, agg.at[dst_s.at[i0 + 1]], add=True)
        return carry

    lax.fori_loop(0, NPAIR16, body, 0)
    pltpu.make_async_copy(table.at[src_s.at[0]], rows0, sem0).wait()
    plsc.subcore_barrier()
    pltpu.sync_copy(agg.at[pl.ds(r0, ROWS_PER_TILE)],
                    out.at[cid, pl.ds(r0, ROWS_PER_TILE)])


def _xw2_split_body(x_ref, w_ref, wl_ref, b_ref, s_ref, l_ref):
    x = x_ref[...]
    s = jnp.dot(x, w_ref[...], preferred_element_type=jnp.float32)
    s_ref[0] = s[:, :HC]
    s_ref[1] = s[:, HC:]
    l_ref[...] = jnp.dot(x, wl_ref[...], preferred_element_type=jnp.float32) + b_ref[...]


def _xw2_split(x, w, wl, b, bn=2000):
    n, cin = x.shape
    co = w.shape[1]
    grid = (n // bn,)
    return pl.pallas_call(
        _xw2_split_body,
        grid=grid,
        in_specs=[
            pl.BlockSpec((bn, cin), lambda i: (i, 0)),
            pl.BlockSpec((cin, co), lambda i: (0, 0)),
            pl.BlockSpec((cin, co), lambda i: (0, 0)),
            pl.BlockSpec((1, co), lambda i: (0, 0)),
        ],
        out_specs=[
            pl.BlockSpec((NC, bn, HC), lambda i: (0, i, 0)),
            pl.BlockSpec((bn, co), lambda i: (i, 0)),
        ],
        out_shape=[
            jax.ShapeDtypeStruct((NC, n, HC), jnp.float32),
            jax.ShapeDtypeStruct((n, co), jnp.float32),
        ],
    )(x, w, wl, b.reshape(1, co))


def _xw2_plain_body(x_ref, w_ref, wl_ref, b_ref, s_ref, l_ref):
    x = x_ref[...]
    s_ref[...] = jnp.dot(x, w_ref[...], preferred_element_type=jnp.float32)
    l_ref[...] = jnp.dot(x, wl_ref[...], preferred_element_type=jnp.float32) + b_ref[...]


def _xw2_plain(x, w, wl, b, bn=2000):
    n, cin = x.shape
    co = w.shape[1]
    grid = (n // bn,)
    return pl.pallas_call(
        _xw2_plain_body,
        grid=grid,
        in_specs=[
            pl.BlockSpec((bn, cin), lambda i: (i, 0)),
            pl.BlockSpec((cin, co), lambda i: (0, 0)),
            pl.BlockSpec((cin, co), lambda i: (0, 0)),
            pl.BlockSpec((1, co), lambda i: (0, 0)),
        ],
        out_specs=[
            pl.BlockSpec((bn, co), lambda i: (i, 0)),
            pl.BlockSpec((bn, co), lambda i: (i, 0)),
        ],
        out_shape=[
            jax.ShapeDtypeStruct((n, co), jnp.float32),
            jax.ShapeDtypeStruct((n, co), jnp.float32),
        ],
    )(x, w, wl, b.reshape(1, co))


def _merge_relu_body(z0_ref, z1_ref, l_ref, o_ref):
    z = jnp.concatenate([z0_ref[...], z1_ref[...]], axis=1)
    o_ref[...] = jnp.maximum(z + l_ref[...], 0.0)


def _merge_res_body(z0_ref, z1_ref, l_ref, h_ref, o_ref):
    z = jnp.concatenate([z0_ref[...], z1_ref[...]], axis=1)
    y = jnp.maximum(z + l_ref[...], 0.0)
    o_ref[...] = (h_ref[...] + y) * 0.5


def _merge_add_body(z0_ref, z1_ref, l_ref, o_ref):
    o_ref[...] = z0_ref[...] + z1_ref[...] + l_ref[...]


def _merge(body, arrs, co, bn=2000):
    n = arrs[0].shape[0]
    grid = (n // bn,)
    return pl.pallas_call(
        body,
        grid=grid,
        in_specs=[pl.BlockSpec((bn, a.shape[1]), lambda i: (i, 0)) for a in arrs],
        out_specs=pl.BlockSpec((bn, co), lambda i: (i, 0)),
        out_shape=jax.ShapeDtypeStruct((n, co), jnp.float32),
    )(*arrs)


def kernel(x3, edge_index, W1, Wl1, b1, Wb, Wlb, bb, W2, Wl2, b2, Wg, Wlg, bg):
    x = x3[0]
    src3 = edge_index[0].reshape(NS, NCH2, CH2)
    dst3 = edge_index[1].reshape(NS, NCH2, CH2)
    src4 = edge_index[0].reshape(NW, NCH16, CH2)
    dst4 = edge_index[1].reshape(NW, NCH16, CH2)
    zeros96 = jnp.zeros((NP, HC), jnp.float32)
    zeros16 = jnp.zeros((NP, 16), jnp.float32)

    def gconv_relu(xin, w, wl, b):
        s2, l = _xw2_split(xin, w, wl, b)
        parts = _segsum192(s2, src3, dst3, zeros96)
        return _merge(_merge_relu_body, (parts[0, :N], parts[1, :N], l), 192)

    def gconv_res(xin, w, wl, b, h):
        s2, l = _xw2_split(xin, w, wl, b)
        parts = _segsum192(s2, src3, dst3, zeros96)
        return _merge(_merge_res_body, (parts[0, :N], parts[1, :N], l, h), 192)

    h = gconv_relu(x, W1, Wl1, b1)
    for i in range(6):
        y = gconv_relu(h, Wb[2 * i], Wlb[2 * i], bb[2 * i])
        h = gconv_res(y, Wb[2 * i + 1], Wlb[2 * i + 1], bb[2 * i + 1], h)
    xo = gconv_relu(h, W2, Wl2, b2)
    wg_p = jnp.pad(Wg, ((0, 0), (0, 13)))
    wlg_p = jnp.pad(Wlg, ((0, 0), (0, 13)))
    bg_p = jnp.pad(bg, (0, 13))
    sg, lg = _xw2_plain(xo, wg_p, wlg_p, bg_p)
    parts = _segsum16(sg, src4, dst4, zeros16)
    out16 = _merge(_merge_add_body, (parts[0, :N], parts[1, :N], lg), 16)
    return out16[:, :3][None]


# speedup vs baseline: 9.4401x; 1.1375x over previous
import functools

import jax
import jax.numpy as jnp
from jax import lax
from jax.experimental import pallas as pl
from jax.experimental.pallas import tpu as pltpu
from jax.experimental.pallas import tpu_sc as plsc

N = 10000
E = 320000
NC = 2
NS = 16
NW = NC * NS
EW = E // NW
NP = 10240
ROWS_PER_TILE = NP // NS

HC = 96
ET = E // NS
CH2 = 125
NCH2 = ET // CH2
NPAIR = NCH2 // 2
NCH16 = EW // CH2
NPAIR16 = NCH16 // 2

_MESH = plsc.VectorSubcoreMesh(core_axis_name="c", subcore_axis_name="s",
                               num_cores=NC, num_subcores=NS)
_SC_PARAMS = pltpu.CompilerParams(use_tc_tiling_on_sc=False)


@functools.partial(
    pl.kernel,
    out_type=jax.ShapeDtypeStruct((NC, NP, HC), jnp.float32),
    mesh=_MESH,
    scratch_types=[
        pltpu.VMEM((2, CH2), jnp.int32),
        pltpu.VMEM((2, CH2), jnp.int32),
        pltpu.VMEM((2, CH2), jnp.int32),
        pltpu.VMEM((2, CH2), jnp.int32),
        pltpu.VMEM((CH2, HC), jnp.float32),
        pltpu.VMEM((CH2, HC), jnp.float32),
        pltpu.VMEM((CH2, HC), jnp.float32),
        pltpu.VMEM((CH2, HC), jnp.float32),
        pltpu.VMEM_SHARED((NP, HC), jnp.float32),
        pltpu.SemaphoreType.DMA,
        pltpu.SemaphoreType.DMA,
        pltpu.SemaphoreType.DMA,
        pltpu.SemaphoreType.DMA,
        pltpu.SemaphoreType.DMA,
        pltpu.SemaphoreType.DMA,
        pltpu.SemaphoreType.DMA,
        pltpu.SemaphoreType.DMA,
    ],
    compiler_params=_SC_PARAMS,
)
def _segsum192(table, ei3, linit, out,
               ib0, ib1, ib2, ib3, rb0, rb1, rb2, rb3, agg,
               is0, is1, is2, is3, gs0, gs1, gs2, gs3):
    cid = lax.axis_index("c")
    sid = lax.axis_index("s")

    r0 = sid * ROWS_PER_TILE
    pltpu.sync_copy(linit.at[cid, pl.ds(r0, ROWS_PER_TILE)],
                    agg.at[pl.ds(r0, ROWS_PER_TILE)])
    plsc.subcore_barrier()

    tab = table.at[cid]
    eidx = ei3.at[sid]
    ibufs = (ib0, ib1, ib2, ib3)
    rbufs = (rb0, rb1, rb2, rb3)
    isems = (is0, is1, is2, is3)
    gsems = (gs0, gs1, gs2, gs3)

    for k in range(4):
        pltpu.async_copy(eidx.at[k], ibufs[k], isems[k])
    for k in range(3):
        pltpu.make_async_copy(eidx.at[0], ibufs[k], isems[k]).wait()
        pltpu.async_copy(tab.at[ibufs[k].at[0]], rbufs[k], gsems[k])

    def body(j, carry):
        i0 = 4 * j
        for k in range(4):
            i = i0 + k
            kn = (k + 3) % 4
            pltpu.make_async_copy(tab.at[ibufs[k].at[0]], rbufs[k],
                                  gsems[k]).wait()
            pltpu.sync_copy(rbufs[k], agg.at[ibufs[k].at[1]], add=True)
            pltpu.async_copy(eidx.at[lax.rem(i + 4, NCH2)],
                             ibufs[k], isems[k])
            pltpu.make_async_copy(eidx.at[0], ibufs[kn], isems[kn]).wait()
            pltpu.async_copy(tab.at[ibufs[kn].at[0]], rbufs[kn],
                             gsems[kn])
        return carry

    lax.fori_loop(0, NCH2 // 4, body, 0)
    for k in range(3):
        pltpu.make_async_copy(tab.at[ibufs[k].at[0]], rbufs[k], gsems[k]).wait()
    pltpu.make_async_copy(eidx.at[0], ibufs[3], isems[3]).wait()
    plsc.subcore_barrier()
    pltpu.sync_copy(agg.at[pl.ds(r0, ROWS_PER_TILE)],
                    out.at[cid, pl.ds(r0, ROWS_PER_TILE)])


@functools.partial(
    pl.kernel,
    out_type=jax.ShapeDtypeStruct((NC, NP, 16), jnp.float32),
    mesh=_MESH,
    scratch_types=[
        pltpu.VMEM((2, CH2), jnp.int32),
        pltpu.VMEM((2, CH2), jnp.int32),
        pltpu.VMEM((2, CH2), jnp.int32),
        pltpu.VMEM((2, CH2), jnp.int32),
        pltpu.VMEM((CH2, 16), jnp.float32),
        pltpu.VMEM((CH2, 16), jnp.float32),
        pltpu.VMEM((CH2, 16), jnp.float32),
        pltpu.VMEM((CH2, 16), jnp.float32),
        pltpu.VMEM_SHARED((NP, 16), jnp.float32),
        pltpu.SemaphoreType.DMA,
        pltpu.SemaphoreType.DMA,
        pltpu.SemaphoreType.DMA,
        pltpu.SemaphoreType.DMA,
        pltpu.SemaphoreType.DMA,
        pltpu.SemaphoreType.DMA,
        pltpu.SemaphoreType.DMA,
        pltpu.SemaphoreType.DMA,
    ],
    compiler_params=_SC_PARAMS,
)
def _segsum16(table, ei4, linit, out,
              ib0, ib1, ib2, ib3, rb0, rb1, rb2, rb3, agg,
              is0, is1, is2, is3, gs0, gs1, gs2, gs3):
    cid = lax.axis_index("c")
    sid = lax.axis_index("s")
    wid = sid * NC + cid

    r0 = sid * ROWS_PER_TILE
    pltpu.sync_copy(linit.at[cid, pl.ds(r0, ROWS_PER_TILE)],
                    agg.at[pl.ds(r0, ROWS_PER_TILE)])
    plsc.subcore_barrier()

    eidx = ei4.at[wid]
    ibufs = (ib0, ib1, ib2, ib3)
    rbufs = (rb0, rb1, rb2, rb3)
    isems = (is0, is1, is2, is3)
    gsems = (gs0, gs1, gs2, gs3)

    for k in range(4):
        pltpu.async_copy(eidx.at[k], ibufs[k], isems[k])
    for k in range(3):
        pltpu.make_async_copy(eidx.at[0], ibufs[k], isems[k]).wait()
        pltpu.async_copy(table.at[ibufs[k].at[0]], rbufs[k], gsems[k])

    def body(j, carry):
        i0 = 4 * j
        for k in range(4):
            i = i0 + k
            kn = (k + 3) % 4
            pltpu.make_async_copy(table.at[ibufs[k].at[0]], rbufs[k],
                                  gsems[k]).wait()
            pltpu.sync_copy(rbufs[k], agg.at[ibufs[k].at[1]], add=True)
            pltpu.async_copy(eidx.at[lax.rem(i + 4, NCH16)],
                             ibufs[k], isems[k])
            pltpu.make_async_copy(eidx.at[0], ibufs[kn], isems[kn]).wait()
            pltpu.async_copy(table.at[ibufs[kn].at[0]], rbufs[kn], gsems[kn])
        return carry

    lax.fori_loop(0, NCH16 // 4, body, 0)
    for k in range(3):
        pltpu.make_async_copy(table.at[ibufs[k].at[0]], rbufs[k], gsems[k]).wait()
    pltpu.make_async_copy(eidx.at[0], ibufs[3], isems[3]).wait()
    plsc.subcore_barrier()
    pltpu.sync_copy(agg.at[pl.ds(r0, ROWS_PER_TILE)],
                    out.at[cid, pl.ds(r0, ROWS_PER_TILE)])


BN = 2000


def _emit_swl(x, w_ref, wl_ref, b_ref, s_ref, l2_ref):
    s = jnp.dot(x, w_ref[...], preferred_element_type=jnp.float32)
    s_ref[0] = s[:, :HC]
    s_ref[1] = s[:, HC:]
    l = jnp.dot(x, wl_ref[...], preferred_element_type=jnp.float32) + b_ref[...]
    l2_ref[0] = l[:, :HC]
    l2_ref[1] = l[:, HC:]


def _head_body(x_ref, w_ref, wl_ref, b_ref, s_ref, l2_ref):
    _emit_swl(x_ref[...], w_ref, wl_ref, b_ref, s_ref, l2_ref)


def _step_relu_body(p_ref, w_ref, wl_ref, b_ref, s_ref, l2_ref, x_ref):
    z = jnp.concatenate([p_ref[0], p_ref[1]], axis=1)
    x = jnp.maximum(z, 0.0)
    x_ref[...] = x
    _emit_swl(x, w_ref, wl_ref, b_ref, s_ref, l2_ref)


def _step_res_body(p_ref, h_ref, w_ref, wl_ref, b_ref,
                   s_ref, l2_ref, x_ref):
    z = jnp.concatenate([p_ref[0], p_ref[1]], axis=1)
    x = (h_ref[...] + jnp.maximum(z, 0.0)) * 0.5
    x_ref[...] = x
    _emit_swl(x, w_ref, wl_ref, b_ref, s_ref, l2_ref)


def _row_spec(c):
    return pl.BlockSpec((BN, c), lambda i: (i, 0))


def _split_spec(c):
    return pl.BlockSpec((NC, BN, c), lambda i: (0, i, 0))


def _wspec(cin, co):
    return pl.BlockSpec((cin, co), lambda i: (0, 0))


def _head(x, w, wl, b):
    n, cin = x.shape
    co = w.shape[1]
    return pl.pallas_call(
        _head_body,
        grid=(n // BN,),
        in_specs=[_row_spec(cin), _wspec(cin, co), _wspec(cin, co),
                  pl.BlockSpec((1, co), lambda i: (0, 0))],
        out_specs=[_split_spec(HC), _split_spec(HC)],
        out_shape=[jax.ShapeDtypeStruct((NC, N, HC), jnp.float32),
                   jax.ShapeDtypeStruct((NC, NP, HC), jnp.float32)],
    )(x, w, wl, b.reshape(1, co))


def _step(parts, h, w, wl, b):
    co = w.shape[1]
    args = [parts] + ([h] if h is not None else []) + [w, wl, b.reshape(1, co)]
    in_specs = [_split_spec(HC)]
    if h is not None:
        in_specs.append(_row_spec(192))
    in_specs += [_wspec(192, co), _wspec(192, co),
                 pl.BlockSpec((1, co), lambda i: (0, 0))]
    return pl.pallas_call(
        _step_res_body if h is not None else _step_relu_body,
        grid=(N // BN,),
        in_specs=in_specs,
        out_specs=[_split_spec(HC), _split_spec(HC), _row_spec(192)],
        out_shape=[jax.ShapeDtypeStruct((NC, N, HC), jnp.float32),
                   jax.ShapeDtypeStruct((NC, NP, HC), jnp.float32),
                   jax.ShapeDtypeStruct((N, 192), jnp.float32)],
    )(*args)


def _stepg_body(p_ref, w_ref, wl_ref, b_ref, s_ref, lg2_ref):
    z = jnp.concatenate([p_ref[0], p_ref[1]], axis=1)
    x = jnp.maximum(z, 0.0)
    s_ref[...] = jnp.dot(x, w_ref[...], preferred_element_type=jnp.float32)
    lg2_ref[0] = jnp.dot(x, wl_ref[...], preferred_element_type=jnp.float32) + b_ref[...]
    lg2_ref[1] = jnp.zeros((x.shape[0], 16), jnp.float32)


def _stepg(parts, wg, wlg, bg):
    return pl.pallas_call(
        _stepg_body,
        grid=(N // BN,),
        in_specs=[_split_spec(HC), _wspec(192, 16),
                  _wspec(192, 16), pl.BlockSpec((1, 16), lambda i: (0, 0))],
        out_specs=[_row_spec(16), _split_spec(16)],
        out_shape=[jax.ShapeDtypeStruct((N, 16), jnp.float32),
                   jax.ShapeDtypeStruct((NC, NP, 16), jnp.float32)],
    )(parts, wg, wlg, bg.reshape(1, 16))


def _add2_body(p_ref, o_ref):
    o_ref[...] = p_ref[0] + p_ref[1]


def _add2(parts):
    return pl.pallas_call(
        _add2_body,
        grid=(N // BN,),
        in_specs=[_split_spec(16)],
        out_specs=_row_spec(16),
        out_shape=jax.ShapeDtypeStruct((N, 16), jnp.float32),
    )(parts)


def kernel(x3, edge_index, W1, Wl1, b1, Wb, Wlb, bb, W2, Wl2, b2, Wg, Wlg, bg):
    x = x3[0]
    ei3 = jnp.transpose(edge_index.reshape(2, NS, NCH2, CH2), (1, 2, 0, 3))
    ei4 = jnp.transpose(edge_index.reshape(2, NW, NCH16, CH2), (1, 2, 0, 3))

    s2, l2 = _head(x, W1, Wl1, b1)
    parts = _segsum192(s2, ei3, l2)

    h = None
    for i in range(13):
        if i == 0:
            w, wl, b = Wb[0], Wlb[0], bb[0]
        elif i < 12:
            w, wl, b = Wb[i], Wlb[i], bb[i]
        else:
            w, wl, b = W2, Wl2, b2
        res = (i % 2 == 0) and (i > 0)
        s2, l2, hx = _step(parts, h if res else None, w, wl, b)
        if i % 2 == 0:
            h = hx
        parts = _segsum192(s2, ei3, l2)

    wg_p = jnp.pad(Wg, ((0, 0), (0, 13)))
    wlg_p = jnp.pad(Wlg, ((0, 0), (0, 13)))
    bg_p = jnp.pad(bg, (0, 13))
    sg, lg2 = _stepg(parts, wg_p, wlg_p, bg_p)
    parts16 = _segsum16(sg, ei4, lg2)
    out16 = _add2(parts16)
    return out16[:, :3][None]
